# Initial kernel scaffold; baseline (speedup 1.0000x reference)
import jax
import jax.numpy as jnp
from jax.experimental import pallas as pl


def kernel(x, feature, threshold, children_left, children_right, value):
    raise NotImplementedError("write your pallas kernel here")


# speedup vs baseline: 2000.9185x; 2000.9185x over previous
import functools

import jax
import jax.numpy as jnp
from jax import lax
from jax.experimental import pallas as pl
from jax.experimental.pallas import tpu as pltpu
from jax.experimental.pallas import tpu_sc as plsc

N_FEATURE = 256
DEPTH = 8
N_INTERNAL = 2**DEPTH - 1
N_LEAF = 2**DEPTH
N_TREE_PAD = 112
N_BATCH = 4096
LANES = 16
NUM_WORKERS = 32
ROWS_PER_W = N_BATCH // NUM_WORKERS


def _tree_kernel_body(x_hbm, feat_hbm, thr_hbm, val_hbm, out_hbm,
                      x_v, feat_v, thr_v, val_v, out_v):
    c = lax.axis_index("c")
    s = lax.axis_index("s")
    wid = s * 2 + c
    base = wid * ROWS_PER_W

    pltpu.sync_copy(x_hbm.at[pl.ds(base, ROWS_PER_W)], x_v)
    pltpu.sync_copy(feat_hbm, feat_v)
    pltpu.sync_copy(thr_hbm, thr_v)
    pltpu.sync_copy(val_hbm, val_v)

    lane = lax.iota(jnp.int32, LANES)

    for bg in range(ROWS_PER_W // LANES):
        b_vec = lane + (bg * LANES)
        acc0 = jnp.zeros((LANES,), jnp.float32)

        def tree_step(t, acc, b_vec=b_vec):
            t_vec = jnp.full((LANES,), t, jnp.int32)
            node = jnp.zeros((LANES,), jnp.int32)
            for _ in range(DEPTH):
                f = plsc.load_gather(feat_v, [t_vec, node])
                th = plsc.load_gather(thr_v, [t_vec, node])
                xv = plsc.load_gather(x_v, [b_vec, f])
                go_right = (xv > th).astype(jnp.int32)
                node = node * 2 + 1 + go_right
            leaf = node - N_INTERNAL
            return acc + plsc.load_gather(val_v, [t_vec, leaf])

        acc = lax.fori_loop(0, N_TREE_PAD, tree_step, acc0)
        out_v[pl.ds(bg * LANES, LANES)] = 1.0 / (1.0 + jnp.exp(-acc))

    pltpu.sync_copy(out_v, out_hbm.at[pl.ds(base, ROWS_PER_W)])


@functools.partial(jax.jit, static_argnames=())
def _run_sc(x, feat_p, thr_p, val_p):
    mesh = plsc.VectorSubcoreMesh(core_axis_name="c", subcore_axis_name="s")
    call = pl.kernel(
        _tree_kernel_body,
        out_type=jax.ShapeDtypeStruct((N_BATCH,), jnp.float32),
        mesh=mesh,
        scratch_types=[
            pltpu.VMEM((ROWS_PER_W, N_FEATURE), jnp.float32),
            pltpu.VMEM((N_TREE_PAD, N_LEAF), jnp.int32),
            pltpu.VMEM((N_TREE_PAD, N_LEAF), jnp.float32),
            pltpu.VMEM((N_TREE_PAD, N_LEAF), jnp.float32),
            pltpu.VMEM((ROWS_PER_W,), jnp.float32),
        ],
        compiler_params=pltpu.CompilerParams(use_tc_tiling_on_sc=False,
                                             needs_layout_passes=False),
    )
    return call(x, feat_p, thr_p, val_p)


def kernel(x, feature, threshold, children_left, children_right, value):
    del children_left, children_right
    n_batch, _ = x.shape
    n_tree, _ = feature.shape
    feat_i = jnp.maximum(feature[:, :N_INTERNAL], 0)
    feat_p = jnp.pad(feat_i, ((0, N_TREE_PAD - n_tree), (0, N_LEAF - N_INTERNAL)))
    thr_p = jnp.pad(threshold[:, :N_INTERNAL],
                    ((0, N_TREE_PAD - n_tree), (0, N_LEAF - N_INTERNAL)))
    val_p = jnp.pad(value[:, N_INTERNAL:, 0], ((0, N_TREE_PAD - n_tree), (0, 0)))
    out = _run_sc(x, feat_p, thr_p, val_p.astype(jnp.float32))
    return out.reshape(n_batch, 1)
